# baseline (speedup 1.0000x reference)
import functools

import jax
import jax.numpy as jnp
from jax import lax
from jax.experimental import pallas as pl
from jax.experimental.pallas import tpu as pltpu
from jax.experimental.pallas import tpu_sc as plsc

L = 16


def kernel(users, items, user_emb, item_emb):
    batch = users.shape[0]
    embed = user_emb.shape[1]
    mesh = plsc.VectorSubcoreMesh(core_axis_name="c", subcore_axis_name="s")
    nw = mesh.num_cores * mesh.num_subcores
    bpw = batch // nw
    n_chunks = bpw // 128
    n_groups = bpw // L
    n_seg = embed // L

    @functools.partial(
        pl.kernel,
        out_type=jax.ShapeDtypeStruct((batch,), jnp.float32),
        mesh=mesh,
        compiler_params=pltpu.CompilerParams(
            needs_layout_passes=False, use_tc_tiling_on_sc=False),
        scratch_types=[
            pltpu.VMEM((n_chunks, 128), jnp.int32),
            pltpu.VMEM((n_chunks, 128), jnp.int32),
            pltpu.VMEM((bpw, embed), jnp.float32),
            pltpu.VMEM((bpw, embed), jnp.float32),
            pltpu.VMEM((bpw,), jnp.float32),
            pltpu.SemaphoreType.DMA,
            pltpu.SemaphoreType.DMA,
        ],
    )
    def mf_kernel(users_hbm, items_hbm, uemb_hbm, iemb_hbm, out_hbm,
                  uidx_v, iidx_v, urows_v, irows_v, out_v, usem, isem):
        wid = lax.axis_index("s") * mesh.num_cores + lax.axis_index("c")
        base = wid * bpw

        for j in range(n_chunks):
            pltpu.sync_copy(users_hbm.at[pl.ds(base + j * 128, 128)], uidx_v.at[j])
            pltpu.sync_copy(items_hbm.at[pl.ds(base + j * 128, 128)], iidx_v.at[j])

        copies = []
        for j in range(n_chunks):
            copies.append(pltpu.async_copy(
                uemb_hbm.at[uidx_v.at[j]], urows_v.at[pl.ds(j * 128, 128)], usem))
            copies.append(pltpu.async_copy(
                iemb_hbm.at[iidx_v.at[j]], irows_v.at[pl.ds(j * 128, 128)], isem))
        for cp in copies:
            cp.wait()

        lane = lax.iota(jnp.int32, L)

        def group_body(g, carry):
            res = jnp.zeros((L,), jnp.float32)
            for j in range(L):
                row = g * L + j
                acc = (urows_v[row, pl.ds(0, L)] * irows_v[row, pl.ds(0, L)])
                for c in range(1, n_seg):
                    acc = acc + (urows_v[row, pl.ds(c * L, L)]
                                 * irows_v[row, pl.ds(c * L, L)])
                res = jnp.where(lane == j, jnp.sum(acc), res)
            out_v[pl.ds(g * L, L)] = res
            return carry

        lax.fori_loop(0, n_groups, group_body, 0)

        pltpu.sync_copy(out_v, out_hbm.at[pl.ds(base, bpw)])

    return mf_kernel(users.astype(jnp.int32), items.astype(jnp.int32),
                     user_emb, item_emb)


# speedup vs baseline: 1.5167x; 1.5167x over previous
import functools

import jax
import jax.numpy as jnp
from jax import lax
from jax.experimental import pallas as pl
from jax.experimental.pallas import tpu as pltpu
from jax.experimental.pallas import tpu_sc as plsc

L = 16
C = 128


def kernel(users, items, user_emb, item_emb):
    batch = users.shape[0]
    embed = user_emb.shape[1]
    mesh = plsc.VectorSubcoreMesh(core_axis_name="c", subcore_axis_name="s")
    nw = mesh.num_cores * mesh.num_subcores
    bpw = batch // nw
    n_chunks = bpw // C

    @functools.partial(
        pl.kernel,
        out_type=jax.ShapeDtypeStruct((batch,), jnp.float32),
        mesh=mesh,
        compiler_params=pltpu.CompilerParams(needs_layout_passes=False),
        scratch_types=[
            pltpu.VMEM((bpw,), jnp.int32),
            pltpu.VMEM((bpw,), jnp.int32),
            pltpu.VMEM((C, embed), jnp.float32),
            pltpu.VMEM((C, embed), jnp.float32),
            pltpu.VMEM((C, embed), jnp.float32),
            pltpu.VMEM((C, embed), jnp.float32),
            pltpu.VMEM((bpw,), jnp.float32),
            pltpu.SemaphoreType.DMA,
            pltpu.SemaphoreType.DMA,
            pltpu.SemaphoreType.DMA,
            pltpu.SemaphoreType.DMA,
        ],
    )
    def mf_kernel(users_hbm, items_hbm, uemb_hbm, iemb_hbm, out_hbm,
                  uidx_v, iidx_v, ubuf0, ubuf1, ibuf0, ibuf1, out_v,
                  su0, su1, si0, si1):
        wid = lax.axis_index("s") * mesh.num_cores + lax.axis_index("c")
        base = wid * bpw

        pltpu.sync_copy(users_hbm.at[pl.ds(base, bpw)], uidx_v)
        pltpu.sync_copy(items_hbm.at[pl.ds(base, bpw)], iidx_v)

        ubufs, ibufs = (ubuf0, ubuf1), (ibuf0, ibuf1)
        usems, isems = (su0, su1), (si0, si1)

        def fire(c):
            ub, ib = ubufs[c % 2], ibufs[c % 2]
            us, ls = usems[c % 2], isems[c % 2]

            def fire_body(g, carry):
                uvec = uidx_v[pl.ds(c * C + g * L, L)]
                ivec = iidx_v[pl.ds(c * C + g * L, L)]
                for j in range(L):
                    row = g * L + j
                    pltpu.async_copy(uemb_hbm.at[uvec[j]], ub.at[row], us)
                    pltpu.async_copy(iemb_hbm.at[ivec[j]], ib.at[row], ls)
                return carry

            lax.fori_loop(0, C // L, fire_body, 0)

        def drain(c):
            pltpu.make_async_copy(uemb_hbm.at[pl.ds(0, C)], ubufs[c % 2],
                                  usems[c % 2]).wait()
            pltpu.make_async_copy(iemb_hbm.at[pl.ds(0, C)], ibufs[c % 2],
                                  isems[c % 2]).wait()

        pos = lax.iota(jnp.int32, L)

        def compute(c):
            ub, ib = ubufs[c % 2], ibufs[c % 2]

            def group_body(g, carry):
                row = g * L + pos
                acc = jnp.zeros((L,), jnp.float32)
                for d in range(embed):
                    d_vec = jnp.full((L,), d, jnp.int32)
                    acc = acc + (plsc.load_gather(ub, [row, d_vec])
                                 * plsc.load_gather(ib, [row, d_vec]))
                out_v[pl.ds(c * C + g * L, L)] = acc
                return carry

            lax.fori_loop(0, C // L, group_body, 0)

        fire(0)
        for c in range(n_chunks):
            if c + 1 < n_chunks:
                fire(c + 1)
            drain(c)
            compute(c)

        pltpu.sync_copy(out_v, out_hbm.at[pl.ds(base, bpw)])

    return mf_kernel(users.astype(jnp.int32), items.astype(jnp.int32),
                     user_emb, item_emb)
